# Initial kernel scaffold; baseline (speedup 1.0000x reference)
import jax
import jax.numpy as jnp
from jax.experimental import pallas as pl


def kernel(h_V, h_E, edge_idx, W1, b1, W2, b2, W3, b3, Wd1, bd1, Wd2, bd2, ln0_w, ln0_b, ln1_w, ln1_b):
    raise NotImplementedError("write your pallas kernel here")


# speedup vs baseline: 4.8599x; 4.8599x over previous
import functools

import jax
import jax.numpy as jnp
from jax import lax
from jax.experimental import pallas as pl
from jax.experimental.pallas import tpu as pltpu
from jax.experimental.pallas import tpu_sc as plsc

_F32 = jnp.float32


def _sc_geometry():
    try:
        info = plsc.get_sparse_core_info()
        return int(info.num_cores), int(info.num_subcores)
    except Exception:
        return 2, 16


def _precompute_nodes(h_V, W1b, W1c, b1, nb):
    n, d = h_V.shape

    def body(hv_ref, wb_ref, wc_ref, b1_ref, a_ref, b_ref):
        hv = hv_ref[...]
        a_ref[...] = jnp.dot(hv, wb_ref[...], preferred_element_type=_F32) + b1_ref[...]
        b_ref[...] = jnp.dot(hv, wc_ref[...], preferred_element_type=_F32)

    return pl.pallas_call(
        body,
        grid=(n // nb,),
        in_specs=[
            pl.BlockSpec((nb, d), lambda i: (i, 0)),
            pl.BlockSpec((d, d), lambda i: (0, 0)),
            pl.BlockSpec((d, d), lambda i: (0, 0)),
            pl.BlockSpec((1, d), lambda i: (0, 0)),
        ],
        out_specs=[
            pl.BlockSpec((nb, d), lambda i: (i, 0)),
            pl.BlockSpec((nb, d), lambda i: (i, 0)),
        ],
        out_shape=[
            jax.ShapeDtypeStruct((n, d), _F32),
            jax.ShapeDtypeStruct((n, d), _F32),
        ],
    )(h_V, W1b, W1c, b1)


def _make_gather(n, e, d, nc, ns, cf):
    nw = nc * ns
    ew = e // nw
    nfull = ew // cf
    tail = ew - nfull * cf
    mesh = plsc.VectorSubcoreMesh(core_axis_name="c", subcore_axis_name="s")

    scratch = [
        pltpu.VMEM((cf,), jnp.int32),
        pltpu.VMEM((cf,), jnp.int32),
        pltpu.VMEM((cf, d), _F32),
        pltpu.VMEM((cf, d), _F32),
        pltpu.SemaphoreType.DMA,
        pltpu.SemaphoreType.DMA,
    ]
    if tail:
        scratch += [
            pltpu.VMEM((tail,), jnp.int32),
            pltpu.VMEM((tail,), jnp.int32),
            pltpu.VMEM((tail, d), _F32),
            pltpu.VMEM((tail, d), _F32),
        ]

    @functools.partial(
        pl.kernel,
        out_type=(
            jax.ShapeDtypeStruct((e, d), _F32),
            jax.ShapeDtypeStruct((e, d), _F32),
        ),
        mesh=mesh,
        scratch_types=scratch,
    )
    def gather_kernel(a_hbm, b_hbm, src_hbm, tgt_hbm, ga_hbm, gb_hbm,
                      sidx, tidx, rows_a, rows_b, sem_a, sem_b, *tails):
        wid = lax.axis_index("s") * nc + lax.axis_index("c")
        base = wid * ew

        def chunk(off, si, ti, ra, rb, m):
            pltpu.sync_copy(src_hbm.at[pl.ds(off, m)], si)
            pltpu.sync_copy(tgt_hbm.at[pl.ds(off, m)], ti)
            ca = pltpu.async_copy(a_hbm.at[si], ra, sem_a)
            cb = pltpu.async_copy(b_hbm.at[ti], rb, sem_b)
            ca.wait()
            cb.wait()
            pltpu.sync_copy(ra, ga_hbm.at[pl.ds(off, m)])
            pltpu.sync_copy(rb, gb_hbm.at[pl.ds(off, m)])

        def body(i, carry):
            chunk(base + i * cf, sidx, tidx, rows_a, rows_b, cf)
            return carry

        lax.fori_loop(0, nfull, body, 0)
        if tail:
            sidx_t, tidx_t, rows_at, rows_bt = tails
            chunk(base + nfull * cf, sidx_t, tidx_t, rows_at, rows_bt, tail)

    return gather_kernel


def _edge_mlp(h_E, ga, gb, W1a, W2, b2, W3, b3, eb):
    e, d = h_E.shape

    def body(he_ref, ga_ref, gb_ref, w1_ref, w2_ref, b2_ref, w3_ref, b3_ref, out_ref):
        x = jnp.dot(he_ref[...], w1_ref[...], preferred_element_type=_F32)
        x = jnp.maximum(x + ga_ref[...] + gb_ref[...], 0.0)
        x = jnp.maximum(jnp.dot(x, w2_ref[...], preferred_element_type=_F32) + b2_ref[...], 0.0)
        out_ref[...] = jnp.dot(x, w3_ref[...], preferred_element_type=_F32) + b3_ref[...]

    blk = lambda i: (i, 0)
    rep = lambda i: (0, 0)
    return pl.pallas_call(
        body,
        grid=(e // eb,),
        in_specs=[
            pl.BlockSpec((eb, d), blk),
            pl.BlockSpec((eb, d), blk),
            pl.BlockSpec((eb, d), blk),
            pl.BlockSpec((d, d), rep),
            pl.BlockSpec((d, d), rep),
            pl.BlockSpec((1, d), rep),
            pl.BlockSpec((d, d), rep),
            pl.BlockSpec((1, d), rep),
        ],
        out_specs=pl.BlockSpec((eb, d), blk),
        out_shape=jax.ShapeDtypeStruct((e, d), _F32),
    )(h_E, ga, gb, W1a, W2, b2, W3, b3)


def _make_scatter(n, e, d, nc, ns, cf):
    nw = nc * ns
    ew = e // nw
    nfull = ew // cf
    tail = ew - nfull * cf
    mesh = plsc.VectorSubcoreMesh(core_axis_name="c", subcore_axis_name="s")

    zrows = 1000
    nz = n // zrows
    crows = 2000
    ncz = n // crows

    scratch = [
        pltpu.VMEM((cf,), jnp.int32),
        pltpu.VMEM((cf, d), _F32),
        pltpu.VMEM((cf,), _F32),
        pltpu.VMEM((crows,), _F32),
        pltpu.VMEM_SHARED((n, d), _F32),
        pltpu.VMEM_SHARED((n,), _F32),
    ]
    if tail:
        scratch += [
            pltpu.VMEM((tail,), jnp.int32),
            pltpu.VMEM((tail, d), _F32),
            pltpu.VMEM((tail,), _F32),
        ]

    @functools.partial(
        pl.kernel,
        out_type=tuple(
            [jax.ShapeDtypeStruct((nc, n, d), _F32)]
            + [jax.ShapeDtypeStruct((n,), _F32) for _ in range(nc)]
        ),
        mesh=mesh,
        scratch_types=scratch,
    )
    def scatter_kernel(msg_hbm, tgt_hbm, znd_hbm, num_hbm, *rest):
        cnt_hbms = rest[:nc]
        tidx, rows, ones, stage, acc, cacc = rest[nc:nc + 6]
        tails = rest[nc + 6:]
        c = lax.axis_index("c")
        s = lax.axis_index("s")
        base = (s * nc + c) * ew

        for j in range(cf // 16):
            ones[pl.ds(j * 16, 16)] = jnp.full((16,), 1.0, _F32)
        for j in range(crows // 16):
            stage[pl.ds(j * 16, 16)] = jnp.zeros((16,), _F32)

        @pl.when(s < nz)
        def _():
            r0 = s * zrows
            pltpu.sync_copy(znd_hbm.at[pl.ds(r0, zrows)], acc.at[pl.ds(r0, zrows)])

        @pl.when(s < ncz)
        def _():
            r0 = s * crows
            pltpu.sync_copy(stage, cacc.at[pl.ds(r0, crows)])

        plsc.subcore_barrier()

        def chunk(off, ti, rw, on, m):
            pltpu.sync_copy(tgt_hbm.at[pl.ds(off, m)], ti)
            pltpu.sync_copy(msg_hbm.at[pl.ds(off, m)], rw)
            pltpu.sync_copy(rw, acc.at[ti], add=True)
            pltpu.sync_copy(on, cacc.at[ti], add=True)

        def body(i, carry):
            chunk(base + i * cf, tidx, rows, ones, cf)
            return carry

        lax.fori_loop(0, nfull, body, 0)
        if tail:
            tidx_t, rows_t, ones_t = tails
            for j in range(tail // 16):
                ones_t[pl.ds(j * 16, 16)] = jnp.full((16,), 1.0, _F32)
            chunk(base + nfull * cf, tidx_t, rows_t, ones_t, tail)

        plsc.subcore_barrier()

        @pl.when(s < nz)
        def _():
            r0 = s * zrows
            pltpu.sync_copy(acc.at[pl.ds(r0, zrows)], num_hbm.at[c, pl.ds(r0, zrows)])

        for k in range(nc):
            @pl.when((s < ncz) & (c == k))
            def _(k=k):
                r0 = s * crows
                pltpu.sync_copy(cacc.at[pl.ds(r0, crows)], stage)
                pltpu.sync_copy(stage, cnt_hbms[k].at[pl.ds(r0, crows)])

    return scatter_kernel


def _node_update(h_V, num, cnt3, Wd1, bd1, Wd2, bd2, ln0_w, ln0_b, ln1_w, ln1_b, nb):
    n, d = h_V.shape
    nc = num.shape[0]
    dh_hid = Wd1.shape[1]

    def body(hv_ref, num_ref, cnt_ref, wd1_ref, bd1_ref, wd2_ref, bd2_ref,
             l0w_ref, l0b_ref, l1w_ref, l1b_ref, out_ref):
        acc = num_ref[0]
        cv = cnt_ref[0]
        for k in range(1, nc):
            acc = acc + num_ref[k]
            cv = cv + cnt_ref[k]
        mask = cv > 0.0
        inv = jnp.where(mask, 1.0 / jnp.where(mask, cv, 1.0), 0.0)
        x = hv_ref[...] + acc * inv

        mu = jnp.mean(x, axis=-1, keepdims=True)
        var = jnp.mean((x - mu) ** 2, axis=-1, keepdims=True)
        x = (x - mu) * jax.lax.rsqrt(var + 1e-5) * l0w_ref[...] + l0b_ref[...]

        h = jnp.maximum(jnp.dot(x, wd1_ref[...], preferred_element_type=_F32) + bd1_ref[...], 0.0)
        y = x + jnp.dot(h, wd2_ref[...], preferred_element_type=_F32) + bd2_ref[...]

        mu2 = jnp.mean(y, axis=-1, keepdims=True)
        var2 = jnp.mean((y - mu2) ** 2, axis=-1, keepdims=True)
        out_ref[...] = (y - mu2) * jax.lax.rsqrt(var2 + 1e-5) * l1w_ref[...] + l1b_ref[...]

    blk = lambda i: (i, 0)
    rep = lambda i: (0, 0)
    return pl.pallas_call(
        body,
        grid=(n // nb,),
        in_specs=[
            pl.BlockSpec((nb, d), blk),
            pl.BlockSpec((nc, nb, d), lambda i: (0, i, 0)),
            pl.BlockSpec((nc, nb, 1), lambda i: (0, i, 0)),
            pl.BlockSpec((d, dh_hid), rep),
            pl.BlockSpec((1, dh_hid), rep),
            pl.BlockSpec((dh_hid, d), rep),
            pl.BlockSpec((1, d), rep),
            pl.BlockSpec((1, d), rep),
            pl.BlockSpec((1, d), rep),
            pl.BlockSpec((1, d), rep),
            pl.BlockSpec((1, d), rep),
        ],
        out_specs=pl.BlockSpec((nb, d), blk),
        out_shape=jax.ShapeDtypeStruct((n, d), _F32),
    )(h_V, num, cnt3, Wd1, bd1, Wd2, bd2, ln0_w, ln0_b, ln1_w, ln1_b)


def kernel(h_V, h_E, edge_idx, W1, b1, W2, b2, W3, b3, Wd1, bd1, Wd2, bd2,
           ln0_w, ln0_b, ln1_w, ln1_b):
    n, d = h_V.shape
    e = h_E.shape[0]
    nc, ns = _sc_geometry()

    src = edge_idx[0]
    tgt = edge_idx[1]
    W1a = W1[:d]
    W1b = W1[d:2 * d]
    W1c = W1[2 * d:]

    A, B = _precompute_nodes(h_V, W1b, W1c, b1.reshape(1, d), nb=2000)

    ga, gb = _make_gather(n, e, d, nc, ns, cf=128)(A, B, src, tgt)

    msg = _edge_mlp(h_E, ga, gb, W1a, W2, b2.reshape(1, d), W3, b3.reshape(1, d), eb=4000)

    znd = jnp.zeros((n, d), _F32)
    num, *cnts = _make_scatter(n, e, d, nc, ns, cf=128)(msg, tgt, znd)
    cnt = jnp.stack(cnts)

    out = _node_update(
        h_V, num, cnt.reshape(nc, n, 1),
        Wd1, bd1.reshape(1, -1), Wd2, bd2.reshape(1, d),
        ln0_w.reshape(1, d), ln0_b.reshape(1, d),
        ln1_w.reshape(1, d), ln1_b.reshape(1, d),
        nb=2000,
    )
    return out


# speedup vs baseline: 7.6274x; 1.5695x over previous
import functools

import jax
import jax.numpy as jnp
from jax import lax
from jax.experimental import pallas as pl
from jax.experimental.pallas import tpu as pltpu
from jax.experimental.pallas import tpu_sc as plsc

_F32 = jnp.float32


def _sc_geometry():
    try:
        info = plsc.get_sparse_core_info()
        return int(info.num_cores), int(info.num_subcores)
    except Exception:
        return 2, 16


def _precompute_nodes(h_V, W1b, W1c, b1, nb):
    n, d = h_V.shape

    def body(hv_ref, wb_ref, wc_ref, b1_ref, a_ref, b_ref):
        hv = hv_ref[...]
        a_ref[...] = jnp.dot(hv, wb_ref[...], preferred_element_type=_F32) + b1_ref[...]
        b_ref[...] = jnp.dot(hv, wc_ref[...], preferred_element_type=_F32)

    return pl.pallas_call(
        body,
        grid=(n // nb,),
        in_specs=[
            pl.BlockSpec((nb, d), lambda i: (i, 0)),
            pl.BlockSpec((d, d), lambda i: (0, 0)),
            pl.BlockSpec((d, d), lambda i: (0, 0)),
            pl.BlockSpec((1, d), lambda i: (0, 0)),
        ],
        out_specs=[
            pl.BlockSpec((nb, d), lambda i: (i, 0)),
            pl.BlockSpec((nb, d), lambda i: (i, 0)),
        ],
        out_shape=[
            jax.ShapeDtypeStruct((n, d), _F32),
            jax.ShapeDtypeStruct((n, d), _F32),
        ],
    )(h_V, W1b, W1c, b1)


def _make_gather(n, e, d, nc, ns, cf):
    nw = nc * ns
    ew = e // nw
    nfull = ew // cf
    tail = ew - nfull * cf
    mesh = plsc.VectorSubcoreMesh(core_axis_name="c", subcore_axis_name="s")

    scratch = [
        pltpu.VMEM((ew,), jnp.int32),
        pltpu.VMEM((ew,), jnp.int32),
        pltpu.VMEM((cf, d), _F32),
        pltpu.VMEM((cf, d), _F32),
        pltpu.VMEM((cf, d), _F32),
        pltpu.VMEM((cf, d), _F32),
        pltpu.SemaphoreType.DMA,
        pltpu.SemaphoreType.DMA,
        pltpu.SemaphoreType.DMA,
        pltpu.SemaphoreType.DMA,
    ]
    if tail:
        scratch += [
            pltpu.VMEM((tail,), jnp.int32),
            pltpu.VMEM((tail,), jnp.int32),
            pltpu.VMEM((tail, d), _F32),
            pltpu.VMEM((tail, d), _F32),
        ]

    @functools.partial(
        pl.kernel,
        out_type=jax.ShapeDtypeStruct((e, d), _F32),
        mesh=mesh,
        scratch_types=scratch,
    )
    def gather_kernel(a_hbm, b_hbm, src_hbm, tgt_hbm, g_hbm,
                      srcall, tgtall, ra0, ra1, rb0, rb1,
                      sg0, sg1, sw0, sw1, *tails):
        wid = lax.axis_index("s") * nc + lax.axis_index("c")
        base = wid * ew
        ras = (ra0, ra1)
        rbs = (rb0, rb1)
        sgs = (sg0, sg1)
        sws = (sw0, sw1)

        pltpu.sync_copy(src_hbm.at[pl.ds(base, ew)], srcall)
        pltpu.sync_copy(tgt_hbm.at[pl.ds(base, ew)], tgtall)

        def accumulate(ra, rb, m):
            def row(r, carry):
                for j in range(d // 16):
                    sl = pl.ds(j * 16, 16)
                    plsc.addupdate(ra.at[r, sl], rb[r, sl])
                return carry
            lax.fori_loop(0, m, row, 0)

        def fire(i, b):
            pltpu.async_copy(a_hbm.at[srcall.at[pl.ds(i * cf, cf)]], ras[b], sgs[b])
            pltpu.async_copy(b_hbm.at[tgtall.at[pl.ds(i * cf, cf)]], rbs[b], sgs[b])

        def drain_gather(b):
            pltpu.make_async_copy(a_hbm.at[pl.ds(0, cf)], ras[b], sgs[b]).wait()
            pltpu.make_async_copy(b_hbm.at[pl.ds(0, cf)], rbs[b], sgs[b]).wait()

        def drain_write(b):
            pltpu.make_async_copy(ras[b], g_hbm.at[pl.ds(0, cf)], sws[b]).wait()

        fire(0, 0)

        def step(i, b):
            @pl.when(i + 1 < nfull)
            def _():
                @pl.when(i >= 1)
                def _():
                    drain_write(1 - b)
                fire(i + 1, 1 - b)

            drain_gather(b)
            accumulate(ras[b], rbs[b], cf)
            pltpu.async_copy(ras[b], g_hbm.at[pl.ds(base + i * cf, cf)], sws[b])

        def outer(g, carry):
            step(2 * g, 0)
            step(2 * g + 1, 1)
            return carry

        lax.fori_loop(0, nfull // 2, outer, 0)
        if nfull % 2:
            step(nfull - 1, 0)
        drain_write(0)
        drain_write(1)

        if tail:
            sidx_t, tidx_t, rows_at, rows_bt = tails
            off = base + nfull * cf
            pltpu.sync_copy(src_hbm.at[pl.ds(off, tail)], sidx_t)
            pltpu.sync_copy(tgt_hbm.at[pl.ds(off, tail)], tidx_t)
            pltpu.async_copy(a_hbm.at[sidx_t], rows_at, sg0).wait()
            pltpu.async_copy(b_hbm.at[tidx_t], rows_bt, sg1).wait()
            accumulate(rows_at, rows_bt, tail)
            pltpu.sync_copy(rows_at, g_hbm.at[pl.ds(off, tail)])

    return gather_kernel


def _edge_mlp(h_E, g, W1a, W2, b2, W3, b3, eb):
    e, d = h_E.shape

    def body(he_ref, g_ref, w1_ref, w2_ref, b2_ref, w3_ref, b3_ref, out_ref):
        x = jnp.dot(he_ref[...], w1_ref[...], preferred_element_type=_F32)
        x = jnp.maximum(x + g_ref[...], 0.0)
        x = jnp.maximum(jnp.dot(x, w2_ref[...], preferred_element_type=_F32) + b2_ref[...], 0.0)
        out_ref[...] = jnp.dot(x, w3_ref[...], preferred_element_type=_F32) + b3_ref[...]

    blk = lambda i: (i, 0)
    rep = lambda i: (0, 0)
    return pl.pallas_call(
        body,
        grid=(e // eb,),
        in_specs=[
            pl.BlockSpec((eb, d), blk),
            pl.BlockSpec((eb, d), blk),
            pl.BlockSpec((d, d), rep),
            pl.BlockSpec((d, d), rep),
            pl.BlockSpec((1, d), rep),
            pl.BlockSpec((d, d), rep),
            pl.BlockSpec((1, d), rep),
        ],
        out_specs=pl.BlockSpec((eb, d), blk),
        out_shape=jax.ShapeDtypeStruct((e, d), _F32),
    )(h_E, g, W1a, W2, b2, W3, b3)


def _make_scatter(n, e, d, nc, ns, cf):
    nw = nc * ns
    ew = e // nw
    nfull = ew // cf
    tail = ew - nfull * cf
    mesh = plsc.VectorSubcoreMesh(core_axis_name="c", subcore_axis_name="s")

    zrows = 1000
    nz = n // zrows
    crows = 2000
    ncz = n // crows

    scratch = [
        pltpu.VMEM((nfull, cf), jnp.int32),
        pltpu.VMEM((cf, d), _F32),
        pltpu.VMEM((cf, d), _F32),
        pltpu.VMEM((cf,), _F32),
        pltpu.VMEM((crows,), _F32),
        pltpu.VMEM_SHARED((n, d), _F32),
        pltpu.VMEM_SHARED((n,), _F32),
        pltpu.SemaphoreType.DMA,
        pltpu.SemaphoreType.DMA,
    ]
    if tail:
        scratch += [
            pltpu.VMEM((tail,), jnp.int32),
            pltpu.VMEM((tail, d), _F32),
            pltpu.VMEM((tail,), _F32),
        ]

    @functools.partial(
        pl.kernel,
        out_type=tuple(
            [jax.ShapeDtypeStruct((nc, n, d), _F32)]
            + [jax.ShapeDtypeStruct((n,), _F32) for _ in range(nc)]
        ),
        mesh=mesh,
        scratch_types=scratch,
    )
    def scatter_kernel(msg_hbm, tgtm_hbm, tgtt_hbm, znd_hbm, num_hbm, *rest):
        cnt_hbms = rest[:nc]
        tgt2, rows0, rows1, ones, stage, acc, cacc, sm0, sm1 = rest[nc:nc + 9]
        tails = rest[nc + 9:]
        c = lax.axis_index("c")
        s = lax.axis_index("s")
        w = s * nc + c
        base = w * ew
        rws = (rows0, rows1)
        sms = (sm0, sm1)

        for j in range(cf // 16):
            ones[pl.ds(j * 16, 16)] = jnp.full((16,), 1.0, _F32)
        for j in range(crows // 16):
            stage[pl.ds(j * 16, 16)] = jnp.zeros((16,), _F32)

        pltpu.sync_copy(tgtm_hbm.at[w], tgt2)

        @pl.when(s < nz)
        def _():
            r0 = s * zrows
            pltpu.sync_copy(znd_hbm.at[pl.ds(r0, zrows)], acc.at[pl.ds(r0, zrows)])

        @pl.when(s < ncz)
        def _():
            r0 = s * crows
            pltpu.sync_copy(stage, cacc.at[pl.ds(r0, crows)])

        plsc.subcore_barrier()

        def fire(i, b):
            pltpu.async_copy(msg_hbm.at[pl.ds(base + i * cf, cf)], rws[b], sms[b])

        def drain(b):
            pltpu.make_async_copy(msg_hbm.at[pl.ds(0, cf)], rws[b], sms[b]).wait()

        fire(0, 0)

        def step(i, b):
            @pl.when(i + 1 < nfull)
            def _():
                fire(i + 1, 1 - b)

            drain(b)
            ti = tgt2.at[i]
            pltpu.sync_copy(rws[b], acc.at[ti], add=True)
            pltpu.sync_copy(ones, cacc.at[ti], add=True)

        def outer(g, carry):
            step(2 * g, 0)
            step(2 * g + 1, 1)
            return carry

        lax.fori_loop(0, nfull // 2, outer, 0)
        if nfull % 2:
            step(nfull - 1, 0)

        if tail:
            tidx_t, rows_t, ones_t = tails
            for j in range(tail // 16):
                ones_t[pl.ds(j * 16, 16)] = jnp.full((16,), 1.0, _F32)
            off = base + nfull * cf
            pltpu.sync_copy(tgtt_hbm.at[pl.ds(w * tail, tail)], tidx_t)
            pltpu.sync_copy(msg_hbm.at[pl.ds(off, tail)], rows_t)
            pltpu.sync_copy(rows_t, acc.at[tidx_t], add=True)
            pltpu.sync_copy(ones_t, cacc.at[tidx_t], add=True)

        plsc.subcore_barrier()

        @pl.when(s < nz)
        def _():
            r0 = s * zrows
            pltpu.sync_copy(acc.at[pl.ds(r0, zrows)], num_hbm.at[c, pl.ds(r0, zrows)])

        for k in range(nc):
            @pl.when((s < ncz) & (c == k))
            def _(k=k):
                r0 = s * crows
                pltpu.sync_copy(cacc.at[pl.ds(r0, crows)], stage)
                pltpu.sync_copy(stage, cnt_hbms[k].at[pl.ds(r0, crows)])

    return scatter_kernel


def _node_update(h_V, num, cnt3, Wd1, bd1, Wd2, bd2, ln0_w, ln0_b, ln1_w, ln1_b, nb):
    n, d = h_V.shape
    nc = num.shape[0]
    dh_hid = Wd1.shape[1]

    def body(hv_ref, num_ref, cnt_ref, wd1_ref, bd1_ref, wd2_ref, bd2_ref,
             l0w_ref, l0b_ref, l1w_ref, l1b_ref, out_ref):
        acc = num_ref[0]
        cv = cnt_ref[0]
        for k in range(1, nc):
            acc = acc + num_ref[k]
            cv = cv + cnt_ref[k]
        mask = cv > 0.0
        inv = jnp.where(mask, 1.0 / jnp.where(mask, cv, 1.0), 0.0)
        x = hv_ref[...] + acc * inv

        mu = jnp.mean(x, axis=-1, keepdims=True)
        var = jnp.mean((x - mu) ** 2, axis=-1, keepdims=True)
        x = (x - mu) * jax.lax.rsqrt(var + 1e-5) * l0w_ref[...] + l0b_ref[...]

        h = jnp.maximum(jnp.dot(x, wd1_ref[...], preferred_element_type=_F32) + bd1_ref[...], 0.0)
        y = x + jnp.dot(h, wd2_ref[...], preferred_element_type=_F32) + bd2_ref[...]

        mu2 = jnp.mean(y, axis=-1, keepdims=True)
        var2 = jnp.mean((y - mu2) ** 2, axis=-1, keepdims=True)
        out_ref[...] = (y - mu2) * jax.lax.rsqrt(var2 + 1e-5) * l1w_ref[...] + l1b_ref[...]

    blk = lambda i: (i, 0)
    rep = lambda i: (0, 0)
    return pl.pallas_call(
        body,
        grid=(n // nb,),
        in_specs=[
            pl.BlockSpec((nb, d), blk),
            pl.BlockSpec((nc, nb, d), lambda i: (0, i, 0)),
            pl.BlockSpec((nc, nb, 1), lambda i: (0, i, 0)),
            pl.BlockSpec((d, dh_hid), rep),
            pl.BlockSpec((1, dh_hid), rep),
            pl.BlockSpec((dh_hid, d), rep),
            pl.BlockSpec((1, d), rep),
            pl.BlockSpec((1, d), rep),
            pl.BlockSpec((1, d), rep),
            pl.BlockSpec((1, d), rep),
            pl.BlockSpec((1, d), rep),
        ],
        out_specs=pl.BlockSpec((nb, d), blk),
        out_shape=jax.ShapeDtypeStruct((n, d), _F32),
    )(h_V, num, cnt3, Wd1, bd1, Wd2, bd2, ln0_w, ln0_b, ln1_w, ln1_b)


def kernel(h_V, h_E, edge_idx, W1, b1, W2, b2, W3, b3, Wd1, bd1, Wd2, bd2,
           ln0_w, ln0_b, ln1_w, ln1_b):
    n, d = h_V.shape
    e = h_E.shape[0]
    nc, ns = _sc_geometry()

    src = edge_idx[0]
    tgt = edge_idx[1]
    W1a = W1[:d]
    W1b = W1[d:2 * d]
    W1c = W1[2 * d:]

    A, B = _precompute_nodes(h_V, W1b, W1c, b1.reshape(1, d), nb=2000)

    g = _make_gather(n, e, d, nc, ns, cf=128)(A, B, src, tgt)

    msg = _edge_mlp(h_E, g, W1a, W2, b2.reshape(1, d), W3, b3.reshape(1, d), eb=4000)

    cf = 128
    nw = nc * ns
    ew = e // nw
    nfull = ew // cf
    tr = tgt.reshape(nw, ew)
    tgt_main = tr[:, :nfull * cf].reshape(nw, nfull, cf)
    tgt_tail = tr[:, nfull * cf:].reshape(-1)
    znd = jnp.zeros((n, d), _F32)
    num, *cnts = _make_scatter(n, e, d, nc, ns, cf=cf)(msg, tgt_main, tgt_tail, znd)
    cnt = jnp.stack(cnts)

    out = _node_update(
        h_V, num, cnt.reshape(nc, n, 1),
        Wd1, bd1.reshape(1, -1), Wd2, bd2.reshape(1, d),
        ln0_w.reshape(1, d), ln0_b.reshape(1, d),
        ln1_w.reshape(1, d), ln1_b.reshape(1, d),
        nb=2000,
    )
    return out


# speedup vs baseline: 7.6436x; 1.0021x over previous
import functools

import jax
import jax.numpy as jnp
import numpy as np
from jax import lax
from jax.experimental import pallas as pl
from jax.experimental.pallas import tpu as pltpu
from jax.experimental.pallas import tpu_sc as plsc

_F32 = jnp.float32


def _sc_geometry():
    try:
        info = plsc.get_sparse_core_info()
        return int(info.num_cores), int(info.num_subcores)
    except Exception:
        return 2, 16


def _precompute_nodes(h_V, W1b, W1c, b1, nb):
    n, d = h_V.shape

    def body(hv_ref, wb_ref, wc_ref, b1_ref, a_ref, b_ref):
        hv = hv_ref[...]
        a_ref[...] = jnp.dot(hv, wb_ref[...], preferred_element_type=_F32) + b1_ref[...]
        b_ref[...] = jnp.dot(hv, wc_ref[...], preferred_element_type=_F32)

    return pl.pallas_call(
        body,
        grid=(n // nb,),
        in_specs=[
            pl.BlockSpec((nb, d), lambda i: (i, 0)),
            pl.BlockSpec((d, d), lambda i: (0, 0)),
            pl.BlockSpec((d, d), lambda i: (0, 0)),
            pl.BlockSpec((1, d), lambda i: (0, 0)),
        ],
        out_specs=[
            pl.BlockSpec((nb, d), lambda i: (i, 0)),
            pl.BlockSpec((nb, d), lambda i: (i, 0)),
        ],
        out_shape=[
            jax.ShapeDtypeStruct((n, d), _F32),
            jax.ShapeDtypeStruct((n, d), _F32),
        ],
    )(h_V, W1b, W1c, b1)


def _make_gather(n, d, nc, ns, estart, ecount, cf):
    nw = nc * ns
    ew = ecount // nw
    nfull = ew // cf
    tail = ew - nfull * cf
    mesh = plsc.VectorSubcoreMesh(core_axis_name="c", subcore_axis_name="s")

    scratch = [
        pltpu.VMEM((ew,), jnp.int32),
        pltpu.VMEM((ew,), jnp.int32),
        pltpu.VMEM((cf, d), _F32),
        pltpu.VMEM((cf, d), _F32),
        pltpu.VMEM((cf, d), _F32),
        pltpu.VMEM((cf, d), _F32),
        pltpu.SemaphoreType.DMA,
        pltpu.SemaphoreType.DMA,
        pltpu.SemaphoreType.DMA,
        pltpu.SemaphoreType.DMA,
    ]
    if tail:
        scratch += [
            pltpu.VMEM((tail,), jnp.int32),
            pltpu.VMEM((tail,), jnp.int32),
            pltpu.VMEM((tail, d), _F32),
            pltpu.VMEM((tail, d), _F32),
        ]

    @functools.partial(
        pl.kernel,
        out_type=jax.ShapeDtypeStruct((ecount, d), _F32),
        mesh=mesh,
        scratch_types=scratch,
    )
    def gather_kernel(a_hbm, b_hbm, src_hbm, tgt_hbm, g_hbm,
                      srcall, tgtall, ra0, ra1, rb0, rb1,
                      sg0, sg1, sw0, sw1, *tails):
        wid = lax.axis_index("s") * nc + lax.axis_index("c")
        base = wid * ew
        ras = (ra0, ra1)
        rbs = (rb0, rb1)
        sgs = (sg0, sg1)
        sws = (sw0, sw1)

        pltpu.sync_copy(src_hbm.at[pl.ds(estart + base, ew)], srcall)
        pltpu.sync_copy(tgt_hbm.at[pl.ds(estart + base, ew)], tgtall)

        def accumulate(ra, rb, m):
            def row(r, carry):
                for j in range(d // 16):
                    sl = pl.ds(j * 16, 16)
                    plsc.addupdate(ra.at[r, sl], rb[r, sl])
                return carry
            lax.fori_loop(0, m, row, 0)

        def fire(i, b):
            pltpu.async_copy(a_hbm.at[srcall.at[pl.ds(i * cf, cf)]], ras[b], sgs[b])
            pltpu.async_copy(b_hbm.at[tgtall.at[pl.ds(i * cf, cf)]], rbs[b], sgs[b])

        def drain_gather(b):
            pltpu.make_async_copy(a_hbm.at[pl.ds(0, cf)], ras[b], sgs[b]).wait()
            pltpu.make_async_copy(b_hbm.at[pl.ds(0, cf)], rbs[b], sgs[b]).wait()

        def drain_write(b):
            pltpu.make_async_copy(ras[b], g_hbm.at[pl.ds(0, cf)], sws[b]).wait()

        fire(0, 0)

        def step(i, b):
            @pl.when(i + 1 < nfull)
            def _():
                @pl.when(i >= 1)
                def _():
                    drain_write(1 - b)
                fire(i + 1, 1 - b)

            drain_gather(b)
            accumulate(ras[b], rbs[b], cf)
            pltpu.async_copy(ras[b], g_hbm.at[pl.ds(base + i * cf, cf)], sws[b])

        def outer(g, carry):
            step(2 * g, 0)
            step(2 * g + 1, 1)
            return carry

        lax.fori_loop(0, nfull // 2, outer, 0)
        if nfull % 2:
            step(nfull - 1, (nfull - 1) % 2)
        drain_write(0)
        drain_write(1)

        if tail:
            sidx_t, tidx_t, rows_at, rows_bt = tails
            off = base + nfull * cf
            pltpu.sync_copy(src_hbm.at[pl.ds(estart + off, tail)], sidx_t)
            pltpu.sync_copy(tgt_hbm.at[pl.ds(estart + off, tail)], tidx_t)
            pltpu.async_copy(a_hbm.at[sidx_t], rows_at, sg0).wait()
            pltpu.async_copy(b_hbm.at[tidx_t], rows_bt, sg1).wait()
            accumulate(rows_at, rows_bt, tail)
            pltpu.sync_copy(rows_at, g_hbm.at[pl.ds(off, tail)])

    return gather_kernel


def _edge_mlp(h_E, g, W1a, W2, b2, W3, b3, eb, blk_off):
    d = h_E.shape[1]
    ecount = g.shape[0]

    def body(he_ref, g_ref, w1_ref, w2_ref, b2_ref, w3_ref, b3_ref, out_ref):
        x = jnp.dot(he_ref[...], w1_ref[...], preferred_element_type=_F32)
        x = jnp.maximum(x + g_ref[...], 0.0)
        x = jnp.maximum(jnp.dot(x, w2_ref[...], preferred_element_type=_F32) + b2_ref[...], 0.0)
        out_ref[...] = jnp.dot(x, w3_ref[...], preferred_element_type=_F32) + b3_ref[...]

    blk = lambda i: (i, 0)
    rep = lambda i: (0, 0)
    return pl.pallas_call(
        body,
        grid=(ecount // eb,),
        in_specs=[
            pl.BlockSpec((eb, d), lambda i: (i + blk_off, 0)),
            pl.BlockSpec((eb, d), blk),
            pl.BlockSpec((d, d), rep),
            pl.BlockSpec((d, d), rep),
            pl.BlockSpec((1, d), rep),
            pl.BlockSpec((d, d), rep),
            pl.BlockSpec((1, d), rep),
        ],
        out_specs=pl.BlockSpec((eb, d), blk),
        out_shape=jax.ShapeDtypeStruct((ecount, d), _F32),
    )(h_E, g, W1a, W2, b2, W3, b3)


def _make_scatter(n, d, nc, ns, ecount, cf):
    nw = nc * ns
    ew = ecount // nw
    nfull = ew // cf
    tail = ew - nfull * cf
    mesh = plsc.VectorSubcoreMesh(core_axis_name="c", subcore_axis_name="s")

    zrows = 1000
    nz = n // zrows
    crows = 2000
    ncz = n // crows

    scratch = [
        pltpu.VMEM((nfull, cf), jnp.int32),
        pltpu.VMEM((cf, d), _F32),
        pltpu.VMEM((cf, d), _F32),
        pltpu.VMEM((cf,), _F32),
        pltpu.VMEM((crows,), _F32),
        pltpu.VMEM_SHARED((n, d), _F32),
        pltpu.VMEM_SHARED((n,), _F32),
        pltpu.SemaphoreType.DMA,
        pltpu.SemaphoreType.DMA,
    ]
    if tail:
        scratch += [
            pltpu.VMEM((tail,), jnp.int32),
            pltpu.VMEM((tail, d), _F32),
        ]

    @functools.partial(
        pl.kernel,
        out_type=tuple(
            [jax.ShapeDtypeStruct((nc, n, d), _F32)]
            + [jax.ShapeDtypeStruct((n,), _F32) for _ in range(nc)]
        ),
        mesh=mesh,
        scratch_types=scratch,
    )
    def scatter_kernel(msg_hbm, tgtm_hbm, tgtt_hbm, znd_hbm, num_hbm, *rest):
        cnt_hbms = rest[:nc]
        tgt2, rows0, rows1, ones, stage, acc, cacc, sm0, sm1 = rest[nc:nc + 9]
        tails = rest[nc + 9:]
        c = lax.axis_index("c")
        s = lax.axis_index("s")
        w = s * nc + c
        base = w * ew
        rws = (rows0, rows1)
        sms = (sm0, sm1)

        for j in range(cf // 16):
            ones[pl.ds(j * 16, 16)] = jnp.full((16,), 1.0, _F32)
        for j in range(crows // 16):
            stage[pl.ds(j * 16, 16)] = jnp.zeros((16,), _F32)

        pltpu.sync_copy(tgtm_hbm.at[w], tgt2)

        @pl.when(s < nz)
        def _():
            r0 = s * zrows
            pltpu.sync_copy(znd_hbm.at[pl.ds(r0, zrows)], acc.at[pl.ds(r0, zrows)])

        @pl.when(s < ncz)
        def _():
            r0 = s * crows
            pltpu.sync_copy(stage, cacc.at[pl.ds(r0, crows)])

        plsc.subcore_barrier()

        def fire(i, b):
            pltpu.async_copy(msg_hbm.at[pl.ds(base + i * cf, cf)], rws[b], sms[b])

        def drain(b):
            pltpu.make_async_copy(msg_hbm.at[pl.ds(0, cf)], rws[b], sms[b]).wait()

        fire(0, 0)

        def step(i, b):
            @pl.when(i + 1 < nfull)
            def _():
                fire(i + 1, 1 - b)

            drain(b)
            ti = tgt2.at[i]
            pltpu.sync_copy(rws[b], acc.at[ti], add=True)
            pltpu.sync_copy(ones, cacc.at[ti], add=True)

        def outer(g, carry):
            step(2 * g, 0)
            step(2 * g + 1, 1)
            return carry

        lax.fori_loop(0, nfull // 2, outer, 0)
        if nfull % 2:
            step(nfull - 1, 0)

        if tail:
            tidx_t, rows_t = tails
            off = base + nfull * cf
            pltpu.sync_copy(tgtt_hbm.at[pl.ds(w * tail, tail)], tidx_t)
            pltpu.sync_copy(msg_hbm.at[pl.ds(off, tail)], rows_t)
            pltpu.sync_copy(rows_t, acc.at[tidx_t], add=True)
            pltpu.sync_copy(ones.at[pl.ds(0, tail)], cacc.at[tidx_t], add=True)

        plsc.subcore_barrier()

        @pl.when(s < nz)
        def _():
            r0 = s * zrows
            pltpu.sync_copy(acc.at[pl.ds(r0, zrows)], num_hbm.at[c, pl.ds(r0, zrows)])

        for k in range(nc):
            @pl.when((s < ncz) & (c == k))
            def _(k=k):
                r0 = s * crows
                pltpu.sync_copy(cacc.at[pl.ds(r0, crows)], stage)
                pltpu.sync_copy(stage, cnt_hbms[k].at[pl.ds(r0, crows)])

    return scatter_kernel


def _node_update(h_V, num, cnt3, Wd1, bd1, Wd2, bd2, ln0_w, ln0_b, ln1_w, ln1_b, nb):
    n, d = h_V.shape
    nparts = num.shape[0]
    dh_hid = Wd1.shape[1]

    def body(hv_ref, num_ref, cnt_ref, wd1_ref, bd1_ref, wd2_ref, bd2_ref,
             l0w_ref, l0b_ref, l1w_ref, l1b_ref, out_ref):
        acc = num_ref[0]
        cv = cnt_ref[0]
        for k in range(1, nparts):
            acc = acc + num_ref[k]
            cv = cv + cnt_ref[k]
        mask = cv > 0.0
        inv = jnp.where(mask, 1.0 / jnp.where(mask, cv, 1.0), 0.0)
        x = hv_ref[...] + acc * inv

        mu = jnp.mean(x, axis=-1, keepdims=True)
        var = jnp.mean((x - mu) ** 2, axis=-1, keepdims=True)
        x = (x - mu) * jax.lax.rsqrt(var + 1e-5) * l0w_ref[...] + l0b_ref[...]

        h = jnp.maximum(jnp.dot(x, wd1_ref[...], preferred_element_type=_F32) + bd1_ref[...], 0.0)
        y = x + jnp.dot(h, wd2_ref[...], preferred_element_type=_F32) + bd2_ref[...]

        mu2 = jnp.mean(y, axis=-1, keepdims=True)
        var2 = jnp.mean((y - mu2) ** 2, axis=-1, keepdims=True)
        out_ref[...] = (y - mu2) * jax.lax.rsqrt(var2 + 1e-5) * l1w_ref[...] + l1b_ref[...]

    blk = lambda i: (i, 0)
    rep = lambda i: (0, 0)
    return pl.pallas_call(
        body,
        grid=(n // nb,),
        in_specs=[
            pl.BlockSpec((nb, d), blk),
            pl.BlockSpec((nparts, nb, d), lambda i: (0, i, 0)),
            pl.BlockSpec((nparts, nb, 1), lambda i: (0, i, 0)),
            pl.BlockSpec((d, dh_hid), rep),
            pl.BlockSpec((1, dh_hid), rep),
            pl.BlockSpec((dh_hid, d), rep),
            pl.BlockSpec((1, d), rep),
            pl.BlockSpec((1, d), rep),
            pl.BlockSpec((1, d), rep),
            pl.BlockSpec((1, d), rep),
            pl.BlockSpec((1, d), rep),
        ],
        out_specs=pl.BlockSpec((nb, d), blk),
        out_shape=jax.ShapeDtypeStruct((n, d), _F32),
    )(h_V, num, cnt3, Wd1, bd1, Wd2, bd2, ln0_w, ln0_b, ln1_w, ln1_b)


def kernel(h_V, h_E, edge_idx, W1, b1, W2, b2, W3, b3, Wd1, bd1, Wd2, bd2,
           ln0_w, ln0_b, ln1_w, ln1_b):
    n, d = h_V.shape
    e = h_E.shape[0]
    nc, ns = _sc_geometry()
    nw = nc * ns
    cf = 128
    eb = 4000
    nhalves = 2
    he = e // nhalves

    src = edge_idx[0]
    tgt = edge_idx[1]
    W1a = W1[:d]
    W1b = W1[d:2 * d]
    W1c = W1[2 * d:]
    b2r = b2.reshape(1, d)
    b3r = b3.reshape(1, d)

    A, B = _precompute_nodes(h_V, W1b, W1c, b1.reshape(1, d), nb=2000)

    ew = he // nw
    nfull = ew // cf
    znd = jnp.zeros((n, d), _F32)

    gather = _make_gather(n, d, nc, ns, 0, he, cf)
    gather2 = _make_gather(n, d, nc, ns, he, he, cf)
    scatter = _make_scatter(n, d, nc, ns, he, cf)

    def half_idx(h):
        tr = lax.dynamic_slice_in_dim(tgt, h * he, he).reshape(nw, ew)
        tm = tr[:, :nfull * cf].reshape(nw, nfull, cf)
        tt = tr[:, nfull * cf:].reshape(-1)
        return tm, tt

    tm0, tt0 = half_idx(0)
    tm1, tt1 = half_idx(1)

    g0 = gather(A, B, src, tgt)
    msg0 = _edge_mlp(h_E, g0, W1a, W2, b2r, W3, b3r, eb, 0)
    g1 = gather2(A, B, src, tgt)
    num0, *cnts0 = scatter(msg0, tm0, tt0, znd)
    msg1 = _edge_mlp(h_E, g1, W1a, W2, b2r, W3, b3r, eb, he // eb)
    num1, *cnts1 = scatter(msg1, tm1, tt1, znd)

    num = jnp.concatenate([num0, num1], axis=0)
    cnt = jnp.stack(cnts0 + cnts1)

    out = _node_update(
        h_V, num, cnt.reshape(2 * nc, n, 1),
        Wd1, bd1.reshape(1, -1), Wd2, bd2.reshape(1, d),
        ln0_w.reshape(1, d), ln0_b.reshape(1, d),
        ln1_w.reshape(1, d), ln1_b.reshape(1, d),
        nb=2000,
    )
    return out
